# baseline (speedup 1.0000x reference)
import jax
import jax.numpy as jnp
from jax.experimental import pallas as pl
from jax.experimental.pallas import tpu as pltpu


def _mm_block(a_ref, w_ref, o_ref):
    @pl.when(pl.program_id(1) == 0)
    def _init():
        o_ref[...] = jnp.zeros_like(o_ref)

    o_ref[...] += jnp.dot(a_ref[...], w_ref[...], preferred_element_type=jnp.float32)


def kernel(explicit_feedback, dense_weight_slice, item_ids):
    m, k = explicit_feedback.shape
    _, n = dense_weight_slice.shape
    w16 = dense_weight_slice
    bm, bk = 512, 1024
    out = pl.pallas_call(
        _mm_block,
        grid=(m // bm, k // bk),
        compiler_params=pltpu.CompilerParams(
            dimension_semantics=("parallel", "arbitrary"),
        ),
        in_specs=[
            pl.BlockSpec((bm, bk), lambda i, j: (i, j)),
            pl.BlockSpec((bk, n), lambda i, j: (j, 0)),
        ],
        out_specs=pl.BlockSpec((bm, n), lambda i, j: (i, 0)),
        out_shape=jax.ShapeDtypeStruct((m, n), jnp.float32),
    )(explicit_feedback, w16)
    return out


# speedup vs baseline: 1.6716x; 1.6716x over previous
import jax
import jax.numpy as jnp
from jax.experimental import pallas as pl
from jax.experimental.pallas import tpu as pltpu


def _mm_block(a0_ref, a1_ref, w_ref, o_ref):
    kh = a0_ref.shape[1]
    acc = jnp.dot(a0_ref[...], w_ref[:kh, :], preferred_element_type=jnp.float32)
    acc += jnp.dot(a1_ref[...], w_ref[kh:, :], preferred_element_type=jnp.float32)
    o_ref[...] = acc


def kernel(explicit_feedback, dense_weight_slice, item_ids):
    m, k = explicit_feedback.shape
    _, n = dense_weight_slice.shape
    bm = 512
    kh = k // 2
    out = pl.pallas_call(
        _mm_block,
        grid=(m // bm,),
        compiler_params=pltpu.CompilerParams(
            dimension_semantics=("parallel",),
        ),
        in_specs=[
            pl.BlockSpec((bm, kh), lambda i: (i, 0)),
            pl.BlockSpec((bm, kh), lambda i: (i, 1)),
            pl.BlockSpec((k, n), lambda i: (0, 0)),
        ],
        out_specs=pl.BlockSpec((bm, n), lambda i: (i, 0)),
        out_shape=jax.ShapeDtypeStruct((m, n), jnp.float32),
    )(explicit_feedback, explicit_feedback, dense_weight_slice)
    return out
